# Initial kernel scaffold; baseline (speedup 1.0000x reference)
import jax
import jax.numpy as jnp
from jax.experimental import pallas as pl


def kernel(x, edge_index, W, attn_l, attn_r, W_res, bias):
    raise NotImplementedError("write your pallas kernel here")


# speedup vs baseline: 4.8659x; 4.8659x over previous
import functools

import jax
import jax.numpy as jnp
from jax.experimental import pallas as pl
from jax.experimental.pallas import tpu as pltpu

N = 10000
E = 160000
D = 256
H = 4
OUT = 256
F = H * OUT

BLK = 512
GRID = (N + BLK - 1) // BLK


def _proj_body(x_ref, w_ref, wres_ref, bias_ref, attn_ref, feat_ref,
               resb_ref, eler_ref):
  xb = x_ref[...]
  feat = jnp.dot(xb, w_ref[...], preferred_element_type=jnp.float32)
  feat_ref[...] = feat
  resb_ref[...] = (
      jnp.dot(xb, wres_ref[...], preferred_element_type=jnp.float32)
      + bias_ref[...])
  eler_ref[...] = jnp.dot(feat, attn_ref[...],
                          preferred_element_type=jnp.float32)


@jax.jit
def _projections(x, W, W_res, bias, attn_mat):
  return pl.pallas_call(
      _proj_body,
      grid=(GRID,),
      in_specs=[
          pl.BlockSpec((BLK, D), lambda i: (i, 0)),
          pl.BlockSpec((D, F), lambda i: (0, 0)),
          pl.BlockSpec((D, F), lambda i: (0, 0)),
          pl.BlockSpec((1, F), lambda i: (0, 0)),
          pl.BlockSpec((F, 2 * H), lambda i: (0, 0)),
      ],
      out_specs=[
          pl.BlockSpec((BLK, F), lambda i: (i, 0)),
          pl.BlockSpec((BLK, F), lambda i: (i, 0)),
          pl.BlockSpec((BLK, 2 * H), lambda i: (i, 0)),
      ],
      out_shape=[
          jax.ShapeDtypeStruct((GRID * BLK, F), jnp.float32),
          jax.ShapeDtypeStruct((GRID * BLK, F), jnp.float32),
          jax.ShapeDtypeStruct((GRID * BLK, 2 * H), jnp.float32),
      ],
  )(x, W, W_res, bias, attn_mat)


def kernel(x, edge_index, W, attn_l, attn_r, W_res, bias):
  src = edge_index[0]
  dst = edge_index[1]

  eye = jnp.eye(H, dtype=jnp.float32)
  al = (attn_l[:, :, None] * eye[:, None, :]).reshape(F, H)
  ar = (attn_r[:, :, None] * eye[:, None, :]).reshape(F, H)
  attn_mat = jnp.concatenate([al, ar], axis=1)

  xp = jnp.pad(x, ((0, GRID * BLK - N), (0, 0)))
  feat_p, resb_p, eler_p = _projections(xp, W, W_res, bias[None, :], attn_mat)
  feat = feat_p[:N]
  resb = resb_p[:N]
  el = eler_p[:N, :H]
  er = eler_p[:N, H:]

  e = el[src] + er[dst]
  e = jax.nn.leaky_relu(e, negative_slope=0.2)
  ex = jnp.exp(e)
  denom = jax.ops.segment_sum(ex, dst, num_segments=N)
  alpha = ex / (denom[dst] + 1e-9)
  msg = feat[src].reshape(E, H, OUT) * alpha[:, :, None]
  out = jax.ops.segment_sum(msg.reshape(E, F), dst, num_segments=N)
  out = out + resb
  return out.reshape(N, H, OUT)


# speedup vs baseline: 12.1386x; 2.4946x over previous
import functools

import jax
import jax.numpy as jnp
from jax import lax
from jax.experimental import pallas as pl
from jax.experimental.pallas import tpu as pltpu
from jax.experimental.pallas import tpu_sc as plsc

N = 10000
E = 160000
D = 256
H = 4
OUT = 256
F = H * OUT

BLK = 512
GRID = (N + BLK - 1) // BLK
NPAD = GRID * BLK

NC = 2
NS = 16

TE1 = 5120
TE1L = E - 31 * TE1
VR1 = TE1 // 16
VR1L = TE1L // 16
NDMA1 = TE1 * H // 128
NDMA1L = TE1L * H // 128

NDW = N * H
NDWP = 40960

RECROWS = E // 16

TE2 = E // NS
VR2 = TE2 // 16
BAT = 32
SUB = 8
CHUNK = 1120
NCHUNK = 9
SPROWS = CHUNK + 16
TRASH = CHUNK + 8
SEL = TE2 + BAT

_SC_PARAMS = pltpu.CompilerParams(needs_layout_passes=False)


def _proj_body(x_ref, w_ref, wres_ref, bias_ref, attn_ref, feat_ref,
               resb_ref, eler_ref):
  xb = x_ref[...]
  feat = jnp.dot(xb, w_ref[...], preferred_element_type=jnp.float32)
  feat_ref[...] = feat
  resb_ref[...] = (
      jnp.dot(xb, wres_ref[...], preferred_element_type=jnp.float32)
      + bias_ref[...])
  eler_ref[...] = jnp.dot(feat, attn_ref[...],
                          preferred_element_type=jnp.float32)


def _projections(x, W, W_res, bias, attn_mat):
  return pl.pallas_call(
      _proj_body,
      grid=(GRID,),
      in_specs=[
          pl.BlockSpec((BLK, D), lambda i: (i, 0)),
          pl.BlockSpec((D, F), lambda i: (0, 0)),
          pl.BlockSpec((D, F), lambda i: (0, 0)),
          pl.BlockSpec((1, F), lambda i: (0, 0)),
          pl.BlockSpec((F, 2 * H), lambda i: (0, 0)),
      ],
      out_specs=[
          pl.BlockSpec((BLK, F), lambda i: (i, 0)),
          pl.BlockSpec((BLK, F), lambda i: (i, 0)),
          pl.BlockSpec((BLK, 2 * H), lambda i: (i, 0)),
      ],
      out_shape=[
          jax.ShapeDtypeStruct((NPAD, F), jnp.float32),
          jax.ShapeDtypeStruct((NPAD, F), jnp.float32),
          jax.ShapeDtypeStruct((NPAD, 2 * H), jnp.float32),
      ],
  )(x, W, W_res, bias, attn_mat)


def _sc_mesh():
  return plsc.VectorSubcoreMesh(
      core_axis_name="c", subcore_axis_name="s",
      num_cores=NC, num_subcores=NS)


def _k1_body(src_hbm, dst_hbm, eler_hbm, z_hbm, ex_hbm, part_hbm,
             eler_v, srcb, dst1d, exflat, idxrow, pbuf, denom_sp):
  c = lax.axis_index("c")
  s = lax.axis_index("s")
  t = s * NC + c
  base = t * TE1
  nv = jnp.where(t < 31, VR1, VR1L)

  pltpu.sync_copy(z_hbm.at[pl.ds(s * 2560, 2560)], pbuf)
  pltpu.sync_copy(pbuf, denom_sp.at[pl.ds(s * 2560, 2560)])

  pltpu.sync_copy(eler_hbm.at[pl.ds(0, N * 2 * H)], eler_v)

  @pl.when(t < 31)
  def _():
    pltpu.sync_copy(src_hbm.at[pl.ds(base, TE1)], srcb)
    pltpu.sync_copy(dst_hbm.at[pl.ds(base, TE1)], dst1d)

  @pl.when(t == 31)
  def _():
    pltpu.sync_copy(src_hbm.at[pl.ds(base, TE1L)], srcb.at[pl.ds(0, TE1L)])
    pltpu.sync_copy(dst_hbm.at[pl.ds(base, TE1L)], dst1d.at[pl.ds(0, TE1L)])

  iota16 = lax.iota(jnp.int32, 16)

  def compute(i, carry):
    off = i * 16
    srcv = srcb[pl.ds(off, 16)]
    dstv = dst1d[pl.ds(off, 16)]
    for h in range(H):
      el = plsc.load_gather(eler_v, [srcv * (2 * H) + h])
      er = plsc.load_gather(eler_v, [dstv * (2 * H) + (h + H)])
      e = el + er
      e = jnp.where(e >= 0.0, e, e * jnp.float32(0.2))
      exv = jnp.exp(e)
      plsc.store_scatter(exflat, [(off + iota16) * H + h], exv)
    return carry
  lax.fori_loop(0, nv, compute, 0)

  @pl.when(t < 31)
  def _():
    pltpu.sync_copy(exflat, ex_hbm.at[pl.ds(base * H, TE1 * H)])

  @pl.when(t == 31)
  def _():
    pltpu.sync_copy(exflat.at[pl.ds(0, TE1L * H)],
                    ex_hbm.at[pl.ds(base * H, TE1L * H)])

  plsc.subcore_barrier()

  def add_j(j, carry):
    for kk in range(2):
      dstv = dst1d[pl.ds(j * 32 + kk * 16, 16)]
      for h in range(H):
        plsc.store_scatter(idxrow, [kk * 64 + iota16 * H + h],
                           dstv * H + h)
    pltpu.sync_copy(exflat.at[pl.ds(j * 128, 128)], denom_sp.at[idxrow],
                    add=True)
    return carry
  lax.fori_loop(0, nv // 2, add_j, 0)

  plsc.subcore_barrier()

  pltpu.sync_copy(denom_sp.at[pl.ds(s * 2560, 2560)], pbuf)
  pltpu.sync_copy(pbuf, part_hbm.at[pl.ds(c * NDWP + s * 2560, 2560)])


def _k1(src, dst, eler_flat, z):
  f = pl.kernel(
      _k1_body,
      out_type=[
          jax.ShapeDtypeStruct((E * H,), jnp.float32),
          jax.ShapeDtypeStruct((NC * NDWP,), jnp.float32),
      ],
      mesh=_sc_mesh(),
      compiler_params=_SC_PARAMS,
      scratch_types=[
          pltpu.VMEM((N * 2 * H,), jnp.float32),
          pltpu.VMEM((TE1,), jnp.int32),
          pltpu.VMEM((TE1,), jnp.int32),
          pltpu.VMEM((TE1 * H,), jnp.float32),
          pltpu.VMEM((128,), jnp.int32),
          pltpu.VMEM((2560,), jnp.float32),
          pltpu.VMEM_SHARED((NDWP,), jnp.float32),
      ],
  )
  return f(src, dst, eler_flat, z)


def _k1b_body(src_hbm, dst_hbm, ex_hbm, part_hbm, rec_hbm,
              denom_v, exb, srcb, dst1d, recbuf, p0, p1):
  c = lax.axis_index("c")
  s = lax.axis_index("s")
  t = s * NC + c
  base = t * TE1
  nv = jnp.where(t < 31, VR1, VR1L)

  def dmerge_full(k, carry):
    pltpu.sync_copy(part_hbm.at[pl.ds(k * 2560, 2560)], p0)
    pltpu.sync_copy(part_hbm.at[pl.ds(NDWP + k * 2560, 2560)], p1)

    def madd(i, carry2):
      denom_v[pl.ds(k * 2560 + i * 16, 16)] = (
          p0[pl.ds(i * 16, 16)] + p1[pl.ds(i * 16, 16)])
      return carry2
    lax.fori_loop(0, 160, madd, 0)
    return carry
  lax.fori_loop(0, NDWP // 2560, dmerge_full, 0)

  @pl.when(t < 31)
  def _():
    pltpu.sync_copy(src_hbm.at[pl.ds(base, TE1)], srcb)
    pltpu.sync_copy(dst_hbm.at[pl.ds(base, TE1)], dst1d)
    pltpu.sync_copy(ex_hbm.at[pl.ds(base * H, TE1 * H)], exb)

  @pl.when(t == 31)
  def _():
    pltpu.sync_copy(src_hbm.at[pl.ds(base, TE1L)], srcb.at[pl.ds(0, TE1L)])
    pltpu.sync_copy(dst_hbm.at[pl.ds(base, TE1L)], dst1d.at[pl.ds(0, TE1L)])
    pltpu.sync_copy(ex_hbm.at[pl.ds(base * H, TE1L * H)],
                    exb.at[pl.ds(0, TE1L * H)])

  iota16 = lax.iota(jnp.int32, 16)
  zf16 = jnp.zeros((16,), jnp.float32)

  def compute(i, carry):
    off = i * 16
    rowv = jnp.full((16,), 0, jnp.int32) + i
    colv = iota16 * 8
    srcv = srcb[pl.ds(off, 16)]
    dstv = dst1d[pl.ds(off, 16)]
    plsc.store_scatter(recbuf, [rowv, colv],
                       plsc.bitcast(srcv, jnp.float32))
    plsc.store_scatter(recbuf, [rowv, colv + 1],
                       plsc.bitcast(dstv, jnp.float32))
    for h in range(H):
      dv = plsc.load_gather(denom_v, [dstv * H + h])
      exv = plsc.load_gather(exb, [(off + iota16) * H + h])
      av = exv / (dv + jnp.float32(1e-9))
      plsc.store_scatter(recbuf, [rowv, colv + (2 + h)], av)
    plsc.store_scatter(recbuf, [rowv, colv + 6], zf16)
    plsc.store_scatter(recbuf, [rowv, colv + 7], zf16)
    return carry
  lax.fori_loop(0, nv, compute, 0)

  @pl.when(t < 31)
  def _():
    pltpu.sync_copy(recbuf, rec_hbm.at[pl.ds(t * VR1, VR1)])

  @pl.when(t == 31)
  def _():
    pltpu.sync_copy(recbuf.at[pl.ds(0, VR1L)],
                    rec_hbm.at[pl.ds(t * VR1, VR1L)])


def _k1b(src, dst, ex, part):
  f = pl.kernel(
      _k1b_body,
      out_type=jax.ShapeDtypeStruct((RECROWS, 128), jnp.float32),
      mesh=_sc_mesh(),
      compiler_params=_SC_PARAMS,
      scratch_types=[
          pltpu.VMEM((NDWP,), jnp.float32),
          pltpu.VMEM((TE1 * H,), jnp.float32),
          pltpu.VMEM((TE1,), jnp.int32),
          pltpu.VMEM((TE1,), jnp.int32),
          pltpu.VMEM((VR1, 128), jnp.float32),
          pltpu.VMEM((2560,), jnp.float32),
          pltpu.VMEM((2560,), jnp.float32),
      ],
  )
  return f(src, dst, ex, part)


def _k2_body(dst_hbm, rec_hbm, feat_hbm, agg_hbm,
             dst_v, eidx_sel, rowbuf, recst, albuf, srcsub0, srcsub1,
             addsub0, addsub1, rrowbuf, colbuf, chunk_sp):
  srcsub = (srcsub0, srcsub1)
  addsub = (addsub0, addsub1)
  c = lax.axis_index("c")
  s = lax.axis_index("s")
  ebase = s * TE2

  pltpu.sync_copy(dst_hbm.at[pl.ds(ebase, TE2)], dst_v)

  iota16 = lax.iota(jnp.int32, 16)
  zi16 = jnp.zeros((16,), jnp.int32)
  zf16 = jnp.zeros((16,), jnp.float32)

  def prefill(i, carry):
    eidx_sel[pl.ds(i * 16, 16)] = zi16
    return carry
  lax.fori_loop(0, SEL // 16, prefill, 0)

  def zero_rowbuf(i, carry):
    r = i // SUB
    col = (i % SUB) * 16
    rowbuf[r, pl.ds(col, 16)] = zf16
    return carry

  def scale_sub(r, carry):
    a = plsc.load_gather(albuf, [zi16 + (((r % SUB) // 2) * BAT + r // SUB)])
    for g in range(8):
      sl = pl.ds(g * 16, 16)
      rowbuf[r, sl] = rowbuf[r, sl] * a
    return carry

  for j in range((NCHUNK + NC - 1) // NC):
    cj = NC * j + c
    rbase = cj * CHUNK

    @pl.when(cj < NCHUNK)
    def _():
      rows_c = jnp.minimum(CHUNK, N - rbase)
      units = rows_c // 8
      u0 = s * units // NS
      u1 = (s + 1) * units // NS

      lax.fori_loop(0, 64 * SUB, zero_rowbuf, 0)

      def zinit(u, carry):
        pltpu.sync_copy(rowbuf.at[pl.ds(0, 64)],
                        chunk_sp.at[pl.ds(u * 64, 64)])
        return carry
      lax.fori_loop(u0, u1, zinit, 0)

      def scan(i, off_v):
        dstv = dst_v[pl.ds(i * 16, 16)]
        m = jnp.logical_and(dstv >= rbase, dstv < rbase + rows_c)
        pos = off_v + plsc.cumsum(m.astype(jnp.int32)) - 1
        plsc.store_scatter(eidx_sel, [pos], i * 16 + iota16, mask=m)
        return off_v + plsc.all_reduce_population_count(m)
      off_v = lax.fori_loop(0, VR2, scan, jnp.zeros((16,), jnp.int32))
      nsel = off_v[0]
      nb = (nsel + BAT - 1) // BAT

      plsc.subcore_barrier()

      def batch(b, carry):
        for kk in range(2):
          eidxv = eidx_sel[pl.ds(b * BAT + kk * 16, 16)]
          rrowbuf[pl.ds(kk * 16, 16)] = (ebase + eidxv) // 16
          colbuf[pl.ds(kk * 16, 16)] = (eidxv % 16) * 8
        pltpu.sync_copy(rec_hbm.at[rrowbuf], recst)
        for kk in range(2):
          kiota = kk * 16 + iota16
          rcol = colbuf[pl.ds(kk * 16, 16)]
          srcv = plsc.bitcast(plsc.load_gather(recst, [kiota, rcol]),
                              jnp.int32)
          dstv2 = plsc.bitcast(plsc.load_gather(recst, [kiota, rcol + 1]),
                               jnp.int32)
          inb = jnp.logical_and(dstv2 >= rbase, dstv2 < rbase + rows_c)
          lane_valid = (b * BAT + kk * 16 + iota16) < nsel
          inb = jnp.logical_and(inb, lane_valid)
          dstlv = jnp.where(inb, dstv2 - rbase, TRASH)
          for k in range(SUB):
            plsc.store_scatter(srcsub[kk], [iota16 * SUB + k],
                               srcv * SUB + k)
            plsc.store_scatter(addsub[kk], [iota16 * SUB + k],
                               dstlv * SUB + k)
          for h in range(H):
            av = plsc.load_gather(recst, [kiota, rcol + (2 + h)])
            albuf[pl.ds(h * BAT + kk * 16, 16)] = av
        pltpu.sync_copy(feat_hbm.at[srcsub[0]], rowbuf.at[pl.ds(0, 128)])
        pltpu.sync_copy(feat_hbm.at[srcsub[1]], rowbuf.at[pl.ds(128, 128)])
        lax.fori_loop(0, BAT * SUB, scale_sub, 0)
        pltpu.sync_copy(rowbuf.at[pl.ds(0, 128)], chunk_sp.at[addsub[0]],
                        add=True)
        pltpu.sync_copy(rowbuf.at[pl.ds(128, 128)], chunk_sp.at[addsub[1]],
                        add=True)
        return carry
      lax.fori_loop(0, nb, batch, 0)

      plsc.subcore_barrier()

      def cout(u, carry):
        pltpu.sync_copy(chunk_sp.at[pl.ds(u * 64, 64)],
                        rowbuf.at[pl.ds(0, 64)])
        pltpu.sync_copy(rowbuf.at[pl.ds(0, 64)],
                        agg_hbm.at[pl.ds((rbase + u * 8) * SUB, 64)])
        return carry
      lax.fori_loop(u0, u1, cout, 0)

      plsc.subcore_barrier()


def _k2(dst, rec, feat):
  f = pl.kernel(
      _k2_body,
      out_type=jax.ShapeDtypeStruct((N * SUB, 128), jnp.float32),
      mesh=_sc_mesh(),
      compiler_params=_SC_PARAMS,
      scratch_types=[
          pltpu.VMEM((TE2,), jnp.int32),
          pltpu.VMEM((SEL,), jnp.int32),
          pltpu.VMEM((BAT * SUB, 128), jnp.float32),
          pltpu.VMEM((BAT, 128), jnp.float32),
          pltpu.VMEM((H * BAT,), jnp.float32),
          pltpu.VMEM((128,), jnp.int32),
          pltpu.VMEM((128,), jnp.int32),
          pltpu.VMEM((128,), jnp.int32),
          pltpu.VMEM((128,), jnp.int32),
          pltpu.VMEM((BAT,), jnp.int32),
          pltpu.VMEM((BAT,), jnp.int32),
          pltpu.VMEM_SHARED((SPROWS * SUB, 128), jnp.float32),
      ],
  )
  return f(dst, rec, feat)


def _k3_body(a_ref, r_ref, o_ref):
  o_ref[...] = a_ref[...] + r_ref[...]


def _k3(agg, resb):
  return pl.pallas_call(
      _k3_body,
      grid=(GRID,),
      in_specs=[
          pl.BlockSpec((BLK * SUB, 128), lambda i: (i, 0)),
          pl.BlockSpec((BLK * SUB, 128), lambda i: (i, 0)),
      ],
      out_specs=pl.BlockSpec((BLK * SUB, 128), lambda i: (i, 0)),
      out_shape=jax.ShapeDtypeStruct((N * SUB, 128), jnp.float32),
  )(agg, resb)


def kernel(x, edge_index, W, attn_l, attn_r, W_res, bias):
  src = edge_index[0]
  dst = edge_index[1]

  eye = jnp.eye(H, dtype=jnp.float32)
  al = (attn_l[:, :, None] * eye[:, None, :]).reshape(F, H)
  ar = (attn_r[:, :, None] * eye[:, None, :]).reshape(F, H)
  attn_mat = jnp.concatenate([al, ar], axis=1)

  xp = jnp.pad(x, ((0, NPAD - N), (0, 0)))
  feat_p, resb_p, eler_p = _projections(xp, W, W_res, bias[None, :], attn_mat)
  eler_flat = eler_p.reshape(NPAD * 2 * H)

  z = jnp.zeros((NDWP,), jnp.float32)
  ex, part = _k1(src, dst, eler_flat, z)
  rec = _k1b(src, dst, ex, part)
  feat_sub = feat_p.reshape(NPAD * SUB, 128)
  agg = _k2(dst, rec, feat_sub)
  resb_sub = resb_p.reshape(NPAD * SUB, 128)[:N * SUB]
  out = _k3(agg, resb_sub)
  return out.reshape(N, H, OUT)


# speedup vs baseline: 15.1883x; 1.2512x over previous
import functools

import jax
import jax.numpy as jnp
from jax import lax
from jax.experimental import pallas as pl
from jax.experimental.pallas import tpu as pltpu
from jax.experimental.pallas import tpu_sc as plsc

N = 10000
E = 160000
D = 256
H = 4
OUT = 256
F = H * OUT

BLK = 512
GRID = (N + BLK - 1) // BLK
NPAD = GRID * BLK

NC = 2
NS = 16

TE1 = 5120
TE1L = E - 31 * TE1
VR1 = TE1 // 16
VR1L = TE1L // 16
NDMA1 = TE1 * H // 128
NDMA1L = TE1L * H // 128

NDW = N * H
NDWP = 40960

RECROWS = E // 16

TE2 = E // NS
VR2 = TE2 // 16
BAT = 16
SUB = 8
CHUNK = 1104
NCHUNK = 10
SPROWS = CHUNK + 16
TRASH = CHUNK + 8
SEL = TE2 + BAT

_SC_PARAMS = pltpu.CompilerParams(needs_layout_passes=False)


def _proj_body(x_ref, w_ref, wres_ref, bias_ref, attn_ref, feat_ref,
               resb_ref, eler_ref):
  xb = x_ref[...]
  feat = jnp.dot(xb, w_ref[...], preferred_element_type=jnp.float32)
  feat_ref[...] = feat
  resb_ref[...] = (
      jnp.dot(xb, wres_ref[...], preferred_element_type=jnp.float32)
      + bias_ref[...])
  eler_ref[...] = jnp.dot(feat, attn_ref[...],
                          preferred_element_type=jnp.float32)


def _projections(x, W, W_res, bias, attn_mat):
  return pl.pallas_call(
      _proj_body,
      grid=(GRID,),
      in_specs=[
          pl.BlockSpec((BLK, D), lambda i: (i, 0)),
          pl.BlockSpec((D, F), lambda i: (0, 0)),
          pl.BlockSpec((D, F), lambda i: (0, 0)),
          pl.BlockSpec((1, F), lambda i: (0, 0)),
          pl.BlockSpec((F, 2 * H), lambda i: (0, 0)),
      ],
      out_specs=[
          pl.BlockSpec((BLK, F), lambda i: (i, 0)),
          pl.BlockSpec((BLK, F), lambda i: (i, 0)),
          pl.BlockSpec((BLK, 2 * H), lambda i: (i, 0)),
      ],
      out_shape=[
          jax.ShapeDtypeStruct((NPAD, F), jnp.float32),
          jax.ShapeDtypeStruct((NPAD, F), jnp.float32),
          jax.ShapeDtypeStruct((NPAD, 2 * H), jnp.float32),
      ],
  )(x, W, W_res, bias, attn_mat)


def _sc_mesh():
  return plsc.VectorSubcoreMesh(
      core_axis_name="c", subcore_axis_name="s",
      num_cores=NC, num_subcores=NS)


def _k1_body(src_hbm, dst_hbm, eler_hbm, z_hbm, ex_hbm, part_hbm,
             eler_v, srcb, dst1d, exflat, idxrow, pbuf, denom_sp):
  c = lax.axis_index("c")
  s = lax.axis_index("s")
  t = s * NC + c
  base = t * TE1
  nv = jnp.where(t < 31, VR1, VR1L)

  pltpu.sync_copy(z_hbm.at[pl.ds(s * 2560, 2560)], pbuf)
  pltpu.sync_copy(pbuf, denom_sp.at[pl.ds(s * 2560, 2560)])

  pltpu.sync_copy(eler_hbm.at[pl.ds(0, N * 2 * H)], eler_v)

  @pl.when(t < 31)
  def _():
    pltpu.sync_copy(src_hbm.at[pl.ds(base, TE1)], srcb)
    pltpu.sync_copy(dst_hbm.at[pl.ds(base, TE1)], dst1d)

  @pl.when(t == 31)
  def _():
    pltpu.sync_copy(src_hbm.at[pl.ds(base, TE1L)], srcb.at[pl.ds(0, TE1L)])
    pltpu.sync_copy(dst_hbm.at[pl.ds(base, TE1L)], dst1d.at[pl.ds(0, TE1L)])

  iota16 = lax.iota(jnp.int32, 16)

  def compute(i, carry):
    off = i * 16
    srcv = srcb[pl.ds(off, 16)]
    dstv = dst1d[pl.ds(off, 16)]
    for h in range(H):
      el = plsc.load_gather(eler_v, [srcv * (2 * H) + h])
      er = plsc.load_gather(eler_v, [dstv * (2 * H) + (h + H)])
      e = el + er
      e = jnp.where(e >= 0.0, e, e * jnp.float32(0.2))
      exv = jnp.exp(e)
      plsc.store_scatter(exflat, [(off + iota16) * H + h], exv)
    return carry
  lax.fori_loop(0, nv, compute, 0)

  @pl.when(t < 31)
  def _():
    pltpu.sync_copy(exflat, ex_hbm.at[pl.ds(base * H, TE1 * H)])

  @pl.when(t == 31)
  def _():
    pltpu.sync_copy(exflat.at[pl.ds(0, TE1L * H)],
                    ex_hbm.at[pl.ds(base * H, TE1L * H)])

  plsc.subcore_barrier()

  def add_j(j, carry):
    for kk in range(2):
      dstv = dst1d[pl.ds(j * 32 + kk * 16, 16)]
      for h in range(H):
        plsc.store_scatter(idxrow, [kk * 64 + iota16 * H + h],
                           dstv * H + h)
    pltpu.sync_copy(exflat.at[pl.ds(j * 128, 128)], denom_sp.at[idxrow],
                    add=True)
    return carry
  lax.fori_loop(0, nv // 2, add_j, 0)

  plsc.subcore_barrier()

  pltpu.sync_copy(denom_sp.at[pl.ds(s * 2560, 2560)], pbuf)
  pltpu.sync_copy(pbuf, part_hbm.at[pl.ds(c * NDWP + s * 2560, 2560)])


def _k1(src, dst, eler_flat, z):
  f = pl.kernel(
      _k1_body,
      out_type=[
          jax.ShapeDtypeStruct((E * H,), jnp.float32),
          jax.ShapeDtypeStruct((NC * NDWP,), jnp.float32),
      ],
      mesh=_sc_mesh(),
      compiler_params=_SC_PARAMS,
      scratch_types=[
          pltpu.VMEM((N * 2 * H,), jnp.float32),
          pltpu.VMEM((TE1,), jnp.int32),
          pltpu.VMEM((TE1,), jnp.int32),
          pltpu.VMEM((TE1 * H,), jnp.float32),
          pltpu.VMEM((128,), jnp.int32),
          pltpu.VMEM((2560,), jnp.float32),
          pltpu.VMEM_SHARED((NDWP,), jnp.float32),
      ],
  )
  return f(src, dst, eler_flat, z)


def _k1b_body(src_hbm, dst_hbm, ex_hbm, part_hbm, rec_hbm,
              denom_v, exb, srcb, dst1d, recbuf, p0, p1):
  c = lax.axis_index("c")
  s = lax.axis_index("s")
  t = s * NC + c
  base = t * TE1
  nv = jnp.where(t < 31, VR1, VR1L)

  def dmerge_full(k, carry):
    pltpu.sync_copy(part_hbm.at[pl.ds(k * 2560, 2560)], p0)
    pltpu.sync_copy(part_hbm.at[pl.ds(NDWP + k * 2560, 2560)], p1)

    def madd(i, carry2):
      denom_v[pl.ds(k * 2560 + i * 16, 16)] = (
          p0[pl.ds(i * 16, 16)] + p1[pl.ds(i * 16, 16)])
      return carry2
    lax.fori_loop(0, 160, madd, 0)
    return carry
  lax.fori_loop(0, NDWP // 2560, dmerge_full, 0)

  @pl.when(t < 31)
  def _():
    pltpu.sync_copy(src_hbm.at[pl.ds(base, TE1)], srcb)
    pltpu.sync_copy(dst_hbm.at[pl.ds(base, TE1)], dst1d)
    pltpu.sync_copy(ex_hbm.at[pl.ds(base * H, TE1 * H)], exb)

  @pl.when(t == 31)
  def _():
    pltpu.sync_copy(src_hbm.at[pl.ds(base, TE1L)], srcb.at[pl.ds(0, TE1L)])
    pltpu.sync_copy(dst_hbm.at[pl.ds(base, TE1L)], dst1d.at[pl.ds(0, TE1L)])
    pltpu.sync_copy(ex_hbm.at[pl.ds(base * H, TE1L * H)],
                    exb.at[pl.ds(0, TE1L * H)])

  iota16 = lax.iota(jnp.int32, 16)
  zf16 = jnp.zeros((16,), jnp.float32)

  def compute(i, carry):
    off = i * 16
    rowv = jnp.full((16,), 0, jnp.int32) + i
    colv = iota16 * 8
    srcv = srcb[pl.ds(off, 16)]
    dstv = dst1d[pl.ds(off, 16)]
    plsc.store_scatter(recbuf, [rowv, colv],
                       plsc.bitcast(srcv, jnp.float32))
    plsc.store_scatter(recbuf, [rowv, colv + 1],
                       plsc.bitcast(dstv, jnp.float32))
    for h in range(H):
      dv = plsc.load_gather(denom_v, [dstv * H + h])
      exv = plsc.load_gather(exb, [(off + iota16) * H + h])
      av = exv / (dv + jnp.float32(1e-9))
      plsc.store_scatter(recbuf, [rowv, colv + (2 + h)], av)
    plsc.store_scatter(recbuf, [rowv, colv + 6], zf16)
    plsc.store_scatter(recbuf, [rowv, colv + 7], zf16)
    return carry
  lax.fori_loop(0, nv, compute, 0)

  @pl.when(t < 31)
  def _():
    pltpu.sync_copy(recbuf, rec_hbm.at[pl.ds(t * VR1, VR1)])

  @pl.when(t == 31)
  def _():
    pltpu.sync_copy(recbuf.at[pl.ds(0, VR1L)],
                    rec_hbm.at[pl.ds(t * VR1, VR1L)])


def _k1b(src, dst, ex, part):
  f = pl.kernel(
      _k1b_body,
      out_type=jax.ShapeDtypeStruct((RECROWS, 128), jnp.float32),
      mesh=_sc_mesh(),
      compiler_params=_SC_PARAMS,
      scratch_types=[
          pltpu.VMEM((NDWP,), jnp.float32),
          pltpu.VMEM((TE1 * H,), jnp.float32),
          pltpu.VMEM((TE1,), jnp.int32),
          pltpu.VMEM((TE1,), jnp.int32),
          pltpu.VMEM((VR1, 128), jnp.float32),
          pltpu.VMEM((2560,), jnp.float32),
          pltpu.VMEM((2560,), jnp.float32),
      ],
  )
  return f(src, dst, ex, part)


def _k2_body(dst_hbm, rec_hbm, feat_hbm, agg_hbm,
             dst_v, eidx_sel, rowbuf0, rowbuf1, recst, albuf0, albuf1,
             srcsub0, srcsub1, addsub0, addsub1, rrowbuf, colbuf,
             gsem0, gsem1, asem0, asem1, chunk_sp):
  rowbuf = (rowbuf0, rowbuf1)
  albuf = (albuf0, albuf1)
  srcsub = (srcsub0, srcsub1)
  addsub = (addsub0, addsub1)
  gsem = (gsem0, gsem1)
  asem = (asem0, asem1)
  c = lax.axis_index("c")
  s = lax.axis_index("s")
  ebase = s * TE2

  pltpu.sync_copy(dst_hbm.at[pl.ds(ebase, TE2)], dst_v)

  iota16 = lax.iota(jnp.int32, 16)
  zi16 = jnp.zeros((16,), jnp.int32)
  zf16 = jnp.zeros((16,), jnp.float32)

  def prefill(i, carry):
    eidx_sel[pl.ds(i * 16, 16)] = zi16
    return carry
  lax.fori_loop(0, SEL // 16, prefill, 0)

  def zero_rowbuf(i, carry):
    r = i // SUB
    col = (i % SUB) * 16
    rowbuf0[r, pl.ds(col, 16)] = zf16
    return carry

  for j in range(NCHUNK // NC):
    cj = NC * j + c
    rbase = cj * CHUNK
    rows_c = jnp.minimum(CHUNK, N - rbase)
    units = rows_c // 8
    u0 = s * units // NS
    u1 = (s + 1) * units // NS

    lax.fori_loop(0, 64 * SUB, zero_rowbuf, 0)

    def zinit(u, carry):
      pltpu.sync_copy(rowbuf0.at[pl.ds(0, 64)],
                      chunk_sp.at[pl.ds(u * 64, 64)])
      return carry
    lax.fori_loop(u0, u1, zinit, 0)

    def scan(i, off_v):
      dstv = dst_v[pl.ds(i * 16, 16)]
      m = jnp.logical_and(dstv >= rbase, dstv < rbase + rows_c)
      pos = off_v + plsc.cumsum(m.astype(jnp.int32)) - 1
      plsc.store_scatter(eidx_sel, [pos], i * 16 + iota16, mask=m)
      return off_v + plsc.all_reduce_population_count(m)
    off_v = lax.fori_loop(0, VR2, scan, jnp.zeros((16,), jnp.int32))
    nsel = off_v[0]
    nb = (nsel + BAT - 1) // BAT

    plsc.subcore_barrier()

    def build(b, par):
      eidxv = eidx_sel[pl.ds(b * BAT, 16)]
      rrowbuf[pl.ds(0, 16)] = (ebase + eidxv) // 16
      colbuf[pl.ds(0, 16)] = (eidxv % 16) * 8
      pltpu.sync_copy(rec_hbm.at[rrowbuf], recst)
      rcol = colbuf[pl.ds(0, 16)]
      srcv = plsc.bitcast(plsc.load_gather(recst, [iota16, rcol]), jnp.int32)
      dstv2 = plsc.bitcast(plsc.load_gather(recst, [iota16, rcol + 1]),
                           jnp.int32)
      inb = jnp.logical_and(dstv2 >= rbase, dstv2 < rbase + rows_c)
      inb = jnp.logical_and(inb, (b * BAT + iota16) < nsel)
      dstlv = jnp.where(inb, dstv2 - rbase, TRASH)
      for k in range(SUB):
        plsc.store_scatter(srcsub[par], [iota16 * SUB + k], srcv * SUB + k)
        plsc.store_scatter(addsub[par], [iota16 * SUB + k], dstlv * SUB + k)
      for h in range(H):
        av = plsc.load_gather(recst, [iota16, rcol + (2 + h)])
        albuf[par][pl.ds(h * BAT, 16)] = av
      pltpu.async_copy(feat_hbm.at[srcsub[par]], rowbuf[par], gsem[par])

    def fin(b, par):
      pltpu.make_async_copy(feat_hbm.at[srcsub[par]], rowbuf[par],
                            gsem[par]).wait()

      def scale_sub(r, carry):
        a = plsc.load_gather(
            albuf[par], [zi16 + (((r % SUB) // 2) * BAT + r // SUB)])
        for g in range(8):
          sl = pl.ds(g * 16, 16)
          rowbuf[par][r, sl] = rowbuf[par][r, sl] * a
        return carry
      lax.fori_loop(0, BAT * SUB, scale_sub, 0)
      pltpu.async_copy(rowbuf[par], chunk_sp.at[addsub[par]], asem[par],
                       add=True)

    def wait_add(par):
      pltpu.make_async_copy(rowbuf[par], chunk_sp.at[addsub[par]],
                            asem[par]).wait()

    @pl.when(nb >= 1)
    def _():
      build(0, 0)

    def batch_pair(b, carry):
      for par in range(2):

        @pl.when(jnp.logical_and(b % 2 == par, b < nb))
        def _():
          @pl.when(b + 1 < nb)
          def _():
            @pl.when(b >= 1)
            def _():
              wait_add(1 - par)
            build(b + 1, 1 - par)
          fin(b, par)
      return carry
    lax.fori_loop(0, nb, batch_pair, 0)

    @pl.when(nb >= 2)
    def _():
      wait_add(0)
      wait_add(1)

    @pl.when(nb == 1)
    def _():
      wait_add(0)

    plsc.subcore_barrier()

    def cout(u, carry):
      pltpu.sync_copy(chunk_sp.at[pl.ds(u * 64, 64)],
                      rowbuf0.at[pl.ds(0, 64)])
      pltpu.sync_copy(rowbuf0.at[pl.ds(0, 64)],
                      agg_hbm.at[pl.ds((rbase + u * 8) * SUB, 64)])
      return carry
    lax.fori_loop(u0, u1, cout, 0)

    plsc.subcore_barrier()


def _k2(dst, rec, feat):
  f = pl.kernel(
      _k2_body,
      out_type=jax.ShapeDtypeStruct((N * SUB, 128), jnp.float32),
      mesh=_sc_mesh(),
      compiler_params=_SC_PARAMS,
      scratch_types=[
          pltpu.VMEM((TE2,), jnp.int32),
          pltpu.VMEM((SEL,), jnp.int32),
          pltpu.VMEM((BAT * SUB, 128), jnp.float32),
          pltpu.VMEM((BAT * SUB, 128), jnp.float32),
          pltpu.VMEM((BAT, 128), jnp.float32),
          pltpu.VMEM((H * BAT,), jnp.float32),
          pltpu.VMEM((H * BAT,), jnp.float32),
          pltpu.VMEM((BAT * SUB,), jnp.int32),
          pltpu.VMEM((BAT * SUB,), jnp.int32),
          pltpu.VMEM((BAT * SUB,), jnp.int32),
          pltpu.VMEM((BAT * SUB,), jnp.int32),
          pltpu.VMEM((BAT,), jnp.int32),
          pltpu.VMEM((BAT,), jnp.int32),
          pltpu.SemaphoreType.DMA,
          pltpu.SemaphoreType.DMA,
          pltpu.SemaphoreType.DMA,
          pltpu.SemaphoreType.DMA,
          pltpu.VMEM_SHARED((SPROWS * SUB, 128), jnp.float32),
      ],
  )
  return f(dst, rec, feat)


def _k3_body(a_ref, r_ref, o_ref):
  o_ref[...] = a_ref[...] + r_ref[...]


def _k3(agg, resb):
  return pl.pallas_call(
      _k3_body,
      grid=(GRID,),
      in_specs=[
          pl.BlockSpec((BLK * SUB, 128), lambda i: (i, 0)),
          pl.BlockSpec((BLK * SUB, 128), lambda i: (i, 0)),
      ],
      out_specs=pl.BlockSpec((BLK * SUB, 128), lambda i: (i, 0)),
      out_shape=jax.ShapeDtypeStruct((N * SUB, 128), jnp.float32),
  )(agg, resb)


def kernel(x, edge_index, W, attn_l, attn_r, W_res, bias):
  src = edge_index[0]
  dst = edge_index[1]

  eye = jnp.eye(H, dtype=jnp.float32)
  al = (attn_l[:, :, None] * eye[:, None, :]).reshape(F, H)
  ar = (attn_r[:, :, None] * eye[:, None, :]).reshape(F, H)
  attn_mat = jnp.concatenate([al, ar], axis=1)

  xp = jnp.pad(x, ((0, NPAD - N), (0, 0)))
  feat_p, resb_p, eler_p = _projections(xp, W, W_res, bias[None, :], attn_mat)
  eler_flat = eler_p.reshape(NPAD * 2 * H)

  z = jnp.zeros((NDWP,), jnp.float32)
  ex, part = _k1(src, dst, eler_flat, z)
  rec = _k1b(src, dst, ex, part)
  feat_sub = feat_p.reshape(NPAD * SUB, 128)
  agg = _k2(dst, rec, feat_sub)
  resb_sub = resb_p.reshape(NPAD * SUB, 128)[:N * SUB]
  out = _k3(agg, resb_sub)
  return out.reshape(N, H, OUT)
